# baseline (speedup 1.0000x reference)
import functools

import jax
import jax.numpy as jnp
from jax import lax
from jax.experimental import pallas as pl
from jax.experimental.pallas import tpu as pltpu
from jax.experimental.pallas import tpu_sc as plsc

NC = 2
NS = 16
NW = NC * NS

IN_ROWS = 1000000
BATCH = 16384
N_FIELDS = 26
OUT_DIM = 64
TOTAL = BATCH * N_FIELDS

TBLK = 128
N_TBLK = IN_ROWS // TBLK
TAIL0 = N_TBLK * TBLK
TPW = (N_TBLK + NW - 1) // NW

CHUNK = 512
N_CHUNKS = TOTAL // CHUNK
CPW = N_CHUNKS // NW
NBUF = 2


def _transpose_body(tt_hbm, tail_hbm, out_hbm, in_bufs, out_bufs,
                    in_sems, out_sems):
    wid = lax.axis_index("s") * NC + lax.axis_index("c")

    def block(k):
        b = wid + NW * k
        return pl.multiple_of(b * TBLK, TBLK), b < N_TBLK

    def start_read(i0, s):
        pltpu.make_async_copy(
            tt_hbm.at[:, pl.ds(i0, TBLK)], in_bufs[s], in_sems[s]
        ).start()

    def wait_read(s):
        pltpu.make_async_copy(
            tt_hbm.at[:, pl.ds(0, TBLK)], in_bufs[s], in_sems[s]
        ).wait()

    def start_write(i0, s):
        row0 = pl.multiple_of(i0 // 2, OUT_DIM)
        pltpu.make_async_copy(
            out_bufs[s], out_hbm.at[pl.ds(row0, OUT_DIM)], out_sems[s]
        ).start()

    def wait_write(s):
        pltpu.make_async_copy(
            out_bufs[s], out_hbm.at[pl.ds(0, OUT_DIM)], out_sems[s]
        ).wait()

    iota16 = lax.iota(jnp.int32, 16)
    rows_q = [iota16 + (16 * q) for q in range(4)]
    cperm = [(iota16 + d) & 15 for d in range(16)]
    rbase = [cperm[d] >> 1 for d in range(16)]
    cbase = [(cperm[d] & 1) * OUT_DIM + iota16 for d in range(16)]

    def transpose(s):
        src = in_bufs[s]
        dst = out_bufs[s]

        def igroup(g, carry):
            for d in range(16):
                cols = cperm[d] + 16 * g
                r = rbase[d] + 8 * g
                for q in range(4):
                    v = plsc.load_gather(src, [rows_q[q], cols])
                    c = cbase[d] + 16 * q
                    plsc.store_scatter(dst, [r, c], v)
            return carry

        lax.fori_loop(0, 8, igroup, 0)

    @pl.when(wid == 0)
    def _():
        pltpu.sync_copy(tail_hbm, out_hbm.at[pl.ds(TAIL0 // 2, 32)])

    for s in range(2):
        i0, ok = block(s)

        @pl.when(ok)
        def _():
            start_read(i0, s)

    def pair(p, carry):
        for s in range(2):
            k = 2 * p + s
            i0, ok = block(k)
            i0n, okn = block(k + 2)

            @pl.when(ok)
            def _():
                wait_read(s)

                @pl.when(k >= 2)
                def _():
                    wait_write(s)

                transpose(s)
                start_write(i0, s)

                @pl.when(okn)
                def _():
                    start_read(i0n, s)
        return carry

    lax.fori_loop(0, (TPW + 1) // 2, pair, 0)

    for s in range(2):
        wait_write(s)


def _gather_body(table_hbm, idx_hbm, out_hbm, idx_v, bufs, sems):
    wid = lax.axis_index("s") * NC + lax.axis_index("c")
    chunk0 = wid * CPW

    pltpu.sync_copy(idx_hbm.at[pl.ds(chunk0, CPW)], idx_v)

    def start_gather(j, b):
        pltpu.make_async_copy(
            table_hbm.at[idx_v.at[j]], bufs[b], sems[b]
        ).start()

    def wait_gather(j, b):
        pltpu.make_async_copy(
            table_hbm.at[idx_v.at[j]], bufs[b], sems[b]
        ).wait()

    def store(j, b):
        row0 = (chunk0 + j) * CHUNK
        pltpu.sync_copy(bufs[b], out_hbm.at[pl.ds(row0, CHUNK)])

    start_gather(0, 0)

    def pair(p, _):
        j = 2 * p
        start_gather(j + 1, 1)
        wait_gather(j, 0)
        store(j, 0)
        start_gather(j + 2, 0)
        wait_gather(j + 1, 1)
        store(j + 1, 1)
        return 0

    lax.fori_loop(0, CPW // 2 - 1, pair, 0)
    start_gather(CPW - 1, 1)
    wait_gather(CPW - 2, 0)
    store(CPW - 2, 0)
    wait_gather(CPW - 1, 1)
    store(CPW - 1, 1)


@jax.jit
def _embed_lookup(batch_flat, emb_t, tail):
    mesh = plsc.VectorSubcoreMesh(core_axis_name="c", subcore_axis_name="s")

    transpose_run = pl.kernel(
        _transpose_body,
        out_type=jax.ShapeDtypeStruct((IN_ROWS // 2, 2 * OUT_DIM), jnp.float32),
        mesh=mesh,
        scratch_types=[
            [pltpu.VMEM((OUT_DIM, TBLK), jnp.float32) for _ in range(2)],
            [pltpu.VMEM((OUT_DIM, TBLK), jnp.float32) for _ in range(2)],
            [pltpu.SemaphoreType.DMA for _ in range(2)],
            [pltpu.SemaphoreType.DMA for _ in range(2)],
        ],
        compiler_params=pltpu.CompilerParams(
            use_tc_tiling_on_sc=True, needs_layout_passes=False
        ),
    )
    table2 = transpose_run(emb_t, tail)
    table = table2.reshape(IN_ROWS, OUT_DIM)

    gather_run = pl.kernel(
        _gather_body,
        out_type=jax.ShapeDtypeStruct((TOTAL, OUT_DIM), jnp.float32),
        mesh=mesh,
        scratch_types=[
            pltpu.VMEM((CPW, CHUNK), jnp.int32),
            [pltpu.VMEM((CHUNK, OUT_DIM), jnp.float32) for _ in range(NBUF)],
            [pltpu.SemaphoreType.DMA for _ in range(NBUF)],
        ],
        compiler_params=pltpu.CompilerParams(use_tc_tiling_on_sc=False),
    )
    return gather_run(table, batch_flat)


def kernel(batch, emb_weight):
    idx = batch.astype(jnp.int32).reshape(N_CHUNKS, CHUNK)
    tail = emb_weight[TAIL0:, :].reshape(32, 2 * OUT_DIM)
    out = _embed_lookup(idx, emb_weight.T, tail)
    return out.reshape(BATCH, N_FIELDS, OUT_DIM)


# speedup vs baseline: 1.5956x; 1.5956x over previous
import functools

import jax
import jax.numpy as jnp
from jax import lax
from jax.experimental import pallas as pl
from jax.experimental.pallas import tpu as pltpu
from jax.experimental.pallas import tpu_sc as plsc

NC = 2
NS = 16
NW = NC * NS

IN_ROWS = 1000000
BATCH = 16384
N_FIELDS = 26
OUT_DIM = 64
TOTAL = BATCH * N_FIELDS

TBLK = 128
N_TBLK = IN_ROWS // TBLK
TAIL0 = N_TBLK * TBLK
TPW = (N_TBLK + NW - 1) // NW

CHUNK = 512
N_CHUNKS = TOTAL // CHUNK
CPW = N_CHUNKS // NW
NBUF = 2


def _transpose_body(tt_hbm, tail_hbm, out_hbm, in_bufs, out_bufs,
                    in_sems, out_sems):
    wid = lax.axis_index("s") * NC + lax.axis_index("c")

    def block(k):
        b = wid + NW * k
        return pl.multiple_of(b * TBLK, TBLK), b < N_TBLK

    def start_read(i0, s):
        pltpu.make_async_copy(
            tt_hbm.at[:, pl.ds(i0, TBLK)], in_bufs[s], in_sems[s]
        ).start()

    def wait_read(s):
        pltpu.make_async_copy(
            tt_hbm.at[:, pl.ds(0, TBLK)], in_bufs[s], in_sems[s]
        ).wait()

    def start_write(i0, s):
        row0 = pl.multiple_of(i0 // 2, OUT_DIM)
        pltpu.make_async_copy(
            out_bufs[s], out_hbm.at[pl.ds(row0, OUT_DIM)], out_sems[s]
        ).start()

    def wait_write(s):
        pltpu.make_async_copy(
            out_bufs[s], out_hbm.at[pl.ds(0, OUT_DIM)], out_sems[s]
        ).wait()

    iota16 = lax.iota(jnp.int32, 16)
    rows_q = [iota16 + (16 * q) for q in range(4)]
    cperm = [(iota16 + d) & 15 for d in range(16)]
    rbase = [cperm[d] >> 1 for d in range(16)]
    cbase = [(cperm[d] & 1) * OUT_DIM + iota16 for d in range(16)]

    def transpose(s):
        src = in_bufs[s]
        dst = out_bufs[s]

        @functools.partial(plsc.parallel_loop, 0, 8)
        def igroup(g):
            for d in range(16):
                cols = cperm[d] + 16 * g
                r = rbase[d] + 8 * g
                for q in range(4):
                    v = plsc.load_gather(src, [rows_q[q], cols])
                    c = cbase[d] + 16 * q
                    plsc.store_scatter(dst, [r, c], v)

    @pl.when(wid == 0)
    def _():
        pltpu.sync_copy(tail_hbm, out_hbm.at[pl.ds(TAIL0 // 2, 32)])

    for s in range(2):
        i0, ok = block(s)

        @pl.when(ok)
        def _():
            start_read(i0, s)

    def pair(p, carry):
        for s in range(2):
            k = 2 * p + s
            i0, ok = block(k)
            i0n, okn = block(k + 2)

            @pl.when(ok)
            def _():
                wait_read(s)

                @pl.when(k >= 2)
                def _():
                    wait_write(s)

                transpose(s)
                start_write(i0, s)

                @pl.when(okn)
                def _():
                    start_read(i0n, s)
        return carry

    lax.fori_loop(0, (TPW + 1) // 2, pair, 0)

    for s in range(2):
        wait_write(s)


def _gather_body(table_hbm, idx_hbm, out_hbm, idx_v, bufs, sems):
    wid = lax.axis_index("s") * NC + lax.axis_index("c")
    chunk0 = wid * CPW

    pltpu.sync_copy(idx_hbm.at[pl.ds(chunk0, CPW)], idx_v)

    def start_gather(j, b):
        pltpu.make_async_copy(
            table_hbm.at[idx_v.at[j]], bufs[b], sems[b]
        ).start()

    def wait_gather(j, b):
        pltpu.make_async_copy(
            table_hbm.at[idx_v.at[j]], bufs[b], sems[b]
        ).wait()

    def store(j, b):
        row0 = (chunk0 + j) * CHUNK
        pltpu.sync_copy(bufs[b], out_hbm.at[pl.ds(row0, CHUNK)])

    start_gather(0, 0)

    def pair(p, _):
        j = 2 * p
        start_gather(j + 1, 1)
        wait_gather(j, 0)
        store(j, 0)
        start_gather(j + 2, 0)
        wait_gather(j + 1, 1)
        store(j + 1, 1)
        return 0

    lax.fori_loop(0, CPW // 2 - 1, pair, 0)
    start_gather(CPW - 1, 1)
    wait_gather(CPW - 2, 0)
    store(CPW - 2, 0)
    wait_gather(CPW - 1, 1)
    store(CPW - 1, 1)


@jax.jit
def _embed_lookup(batch_flat, emb_t, tail):
    mesh = plsc.VectorSubcoreMesh(core_axis_name="c", subcore_axis_name="s")

    transpose_run = pl.kernel(
        _transpose_body,
        out_type=jax.ShapeDtypeStruct((IN_ROWS // 2, 2 * OUT_DIM), jnp.float32),
        mesh=mesh,
        scratch_types=[
            [pltpu.VMEM((OUT_DIM, TBLK), jnp.float32) for _ in range(2)],
            [pltpu.VMEM((OUT_DIM, TBLK), jnp.float32) for _ in range(2)],
            [pltpu.SemaphoreType.DMA for _ in range(2)],
            [pltpu.SemaphoreType.DMA for _ in range(2)],
        ],
        compiler_params=pltpu.CompilerParams(
            use_tc_tiling_on_sc=True, needs_layout_passes=False
        ),
    )
    table2 = transpose_run(emb_t, tail)
    table = table2.reshape(IN_ROWS, OUT_DIM)

    gather_run = pl.kernel(
        _gather_body,
        out_type=jax.ShapeDtypeStruct((TOTAL, OUT_DIM), jnp.float32),
        mesh=mesh,
        scratch_types=[
            pltpu.VMEM((CPW, CHUNK), jnp.int32),
            [pltpu.VMEM((CHUNK, OUT_DIM), jnp.float32) for _ in range(NBUF)],
            [pltpu.SemaphoreType.DMA for _ in range(NBUF)],
        ],
        compiler_params=pltpu.CompilerParams(use_tc_tiling_on_sc=False),
    )
    return gather_run(table, batch_flat)


def kernel(batch, emb_weight):
    idx = batch.astype(jnp.int32).reshape(N_CHUNKS, CHUNK)
    tail = emb_weight[TAIL0:, :].reshape(32, 2 * OUT_DIM)
    out = _embed_lookup(idx, emb_weight.T, tail)
    return out.reshape(BATCH, N_FIELDS, OUT_DIM)


# speedup vs baseline: 1.6328x; 1.0233x over previous
import functools

import jax
import jax.numpy as jnp
from jax import lax
from jax.experimental import pallas as pl
from jax.experimental.pallas import tpu as pltpu
from jax.experimental.pallas import tpu_sc as plsc

NC = 2
NS = 16
NW = NC * NS

IN_ROWS = 1000000
BATCH = 16384
N_FIELDS = 26
OUT_DIM = 64
TOTAL = BATCH * N_FIELDS

TBLK = 128
N_TBLK = IN_ROWS // TBLK
TAIL0 = N_TBLK * TBLK
TPW = (N_TBLK + NW - 1) // NW

CHUNK = 512
N_CHUNKS = TOTAL // CHUNK
CPW = N_CHUNKS // NW
NBUF = 2


def _transpose_body(tt_hbm, tail_hbm, out_hbm, in_bufs, out_bufs,
                    in_sems, out_sems, fence_sem):
    wid = lax.axis_index("s") * NC + lax.axis_index("c")

    def block(k):
        b = wid + NW * k
        return pl.multiple_of(b * TBLK, TBLK), b < N_TBLK

    def start_read(i0, s):
        pltpu.make_async_copy(
            tt_hbm.at[:, pl.ds(i0, TBLK)], in_bufs[s], in_sems[s]
        ).start()

    def wait_read(s):
        pltpu.make_async_copy(
            tt_hbm.at[:, pl.ds(0, TBLK)], in_bufs[s], in_sems[s]
        ).wait()

    def start_write(i0, s):
        row0 = pl.multiple_of(i0 // 2, OUT_DIM)
        pltpu.make_async_copy(
            out_bufs[s], out_hbm.at[pl.ds(row0, OUT_DIM)], out_sems[s]
        ).start()

    def wait_write(s):
        pltpu.make_async_copy(
            out_bufs[s], out_hbm.at[pl.ds(0, OUT_DIM)], out_sems[s]
        ).wait()

    iota16 = lax.iota(jnp.int32, 16)
    rows_q = [iota16 + (16 * q) for q in range(4)]
    cperm = [(iota16 + d) & 15 for d in range(16)]
    rbase = [cperm[d] >> 1 for d in range(16)]
    cbase = [(cperm[d] & 1) * OUT_DIM + iota16 for d in range(16)]

    def transpose(s):
        src = in_bufs[s]
        dst = out_bufs[s]

        @functools.partial(plsc.parallel_loop, 0, 8)
        def igroup(g):
            for d in range(16):
                cols = cperm[d] + 16 * g
                r = rbase[d] + 8 * g
                for q in range(4):
                    v = plsc.load_gather(src, [rows_q[q], cols])
                    c = cbase[d] + 16 * q
                    plsc.store_scatter(dst, [r, c], v)

    @pl.when(wid == 0)
    def _():
        pltpu.sync_copy(tail_hbm, out_hbm.at[pl.ds(TAIL0 // 2, 32)])

    for s in range(4):
        i0, ok = block(s)

        @pl.when(ok)
        def _():
            start_read(i0, s)

    def fence():
        pltpu.semaphore_signal(fence_sem, 1)
        pltpu.semaphore_wait(fence_sem, 1)

    def quad(p, carry):
        for s in range(4):
            k = 4 * p + s
            i0, ok = block(k)
            i0n, okn = block(k + 4)

            @pl.when(ok)
            def _():
                wait_read(s)

                @pl.when(k >= 4)
                def _():
                    wait_write(s)

                fence()
                transpose(s)
                fence()
                start_write(i0, s)

            @pl.when(okn)
            def _():
                start_read(i0n, s)
        return carry

    lax.fori_loop(0, (TPW + 3) // 4, quad, 0)

    for s in range(4):
        wait_write(s)


def _gather_body(table_hbm, idx_hbm, out_hbm, idx_v, bufs, sems):
    wid = lax.axis_index("s") * NC + lax.axis_index("c")
    chunk0 = wid * CPW

    pltpu.sync_copy(idx_hbm.at[pl.ds(chunk0, CPW)], idx_v)

    def start_gather(j, b):
        pltpu.make_async_copy(
            table_hbm.at[idx_v.at[j]], bufs[b], sems[b]
        ).start()

    def wait_gather(j, b):
        pltpu.make_async_copy(
            table_hbm.at[idx_v.at[j]], bufs[b], sems[b]
        ).wait()

    def store(j, b):
        row0 = (chunk0 + j) * CHUNK
        pltpu.sync_copy(bufs[b], out_hbm.at[pl.ds(row0, CHUNK)])

    start_gather(0, 0)

    def pair(p, _):
        j = 2 * p
        start_gather(j + 1, 1)
        wait_gather(j, 0)
        store(j, 0)
        start_gather(j + 2, 0)
        wait_gather(j + 1, 1)
        store(j + 1, 1)
        return 0

    lax.fori_loop(0, CPW // 2 - 1, pair, 0)
    start_gather(CPW - 1, 1)
    wait_gather(CPW - 2, 0)
    store(CPW - 2, 0)
    wait_gather(CPW - 1, 1)
    store(CPW - 1, 1)


@jax.jit
def _embed_lookup(batch_flat, emb_t, tail):
    mesh = plsc.VectorSubcoreMesh(core_axis_name="c", subcore_axis_name="s")

    transpose_run = pl.kernel(
        _transpose_body,
        out_type=jax.ShapeDtypeStruct((IN_ROWS // 2, 2 * OUT_DIM), jnp.float32),
        mesh=mesh,
        scratch_types=[
            [pltpu.VMEM((OUT_DIM, TBLK), jnp.float32) for _ in range(4)],
            [pltpu.VMEM((OUT_DIM, TBLK), jnp.float32) for _ in range(4)],
            [pltpu.SemaphoreType.DMA for _ in range(4)],
            [pltpu.SemaphoreType.DMA for _ in range(4)],
            pltpu.SemaphoreType.REGULAR,
        ],
        compiler_params=pltpu.CompilerParams(
            use_tc_tiling_on_sc=True, needs_layout_passes=False
        ),
    )
    table2 = transpose_run(emb_t, tail)
    table = table2.reshape(IN_ROWS, OUT_DIM)

    gather_run = pl.kernel(
        _gather_body,
        out_type=jax.ShapeDtypeStruct((TOTAL, OUT_DIM), jnp.float32),
        mesh=mesh,
        scratch_types=[
            pltpu.VMEM((CPW, CHUNK), jnp.int32),
            [pltpu.VMEM((CHUNK, OUT_DIM), jnp.float32) for _ in range(NBUF)],
            [pltpu.SemaphoreType.DMA for _ in range(NBUF)],
        ],
        compiler_params=pltpu.CompilerParams(use_tc_tiling_on_sc=False),
    )
    return gather_run(table, batch_flat)


def kernel(batch, emb_weight):
    idx = batch.astype(jnp.int32).reshape(N_CHUNKS, CHUNK)
    tail = emb_weight[TAIL0:, :].reshape(32, 2 * OUT_DIM)
    out = _embed_lookup(idx, emb_weight.T, tail)
    return out.reshape(BATCH, N_FIELDS, OUT_DIM)
